# Initial kernel scaffold; baseline (speedup 1.0000x reference)
import jax
import jax.numpy as jnp
from jax.experimental import pallas as pl


def kernel(x0, init_rel, W, w_rel, basis_freq, phase, W_proj, b_proj, edge_index, edge_type, edge_time, quals, sub, rel, time):
    raise NotImplementedError("write your pallas kernel here")


# speedup vs baseline: 3.2823x; 3.2823x over previous
import functools

import jax
import jax.numpy as jnp
from jax import lax
from jax.experimental import pallas as pl
from jax.experimental.pallas import tpu as pltpu
from jax.experimental.pallas import tpu_sc as plsc

N = 10000
E = 320000
D = 128
NREL = 1000
NQ = 64000
B = 4096
TPAD = 368

NC = 2
NS = 16
NW = NC * NS
L = 16
VPR = D // L

EK = 80
E_PER_W = E // NW
E_CHUNKS = E_PER_W // EK
Q_PER_W = NQ // NW
Q_CHUNKS = Q_PER_W // EK
B_PER_W = B // NW
BK = 64
B_CHUNKS = B_PER_W // BK
NPAD = 10240
N_PER_S = NPAD // NS
ZK = 80
Z_CHUNKS = N_PER_S // ZK

def _mesh():
    return plsc.VectorSubcoreMesh(core_axis_name="c", subcore_axis_name="s",
                                  num_cores=NC, num_subcores=NS)


def _rows_op(dst_ref, k, body):
    def step(t, _):
        i = t // VPR
        j = (t % VPR) * L
        dst_ref[i, pl.ds(j, L)] = body(i, j)
        return 0
    lax.fori_loop(0, k * VPR, step, 0)


def _sc_agg_body(x0_hbm, rel_hbm, tt_hbm, src_hbm, dst_hbm, et_hbm, etime_hbm,
                 qr_hbm, qe_hbm, qedge_hbm,
                 p0_hbm, p1_hbm,
                 i0, i1, i2, i3, i4,
                 abuf, bbuf, tbuf, mbuf,
                 aggS, sem0, sem1, sem2):
    c = lax.axis_index("c")
    s = lax.axis_index("s")
    w = c * NS + s

    zero16 = jnp.zeros((L,), jnp.float32)
    _rows_op(mbuf, ZK, lambda i, j: zero16)
    r0 = s * N_PER_S

    def _zc(k, _):
        pltpu.sync_copy(mbuf, aggS.at[pl.ds(r0 + k * ZK, ZK)])
        return 0
    lax.fori_loop(0, Z_CHUNKS, _zc, 0)
    plsc.subcore_barrier()

    e0 = w * E_PER_W

    def _echunk(ci, _):
        base = e0 + ci * EK
        pltpu.sync_copy(src_hbm.at[pl.ds(base, EK)], i0)
        pltpu.sync_copy(et_hbm.at[pl.ds(base, EK)], i1)
        pltpu.sync_copy(etime_hbm.at[pl.ds(base, EK)], i2)
        pltpu.sync_copy(dst_hbm.at[pl.ds(base, EK)], i3)
        cp1 = pltpu.async_copy(x0_hbm.at[i0], abuf, sem0)
        cp2 = pltpu.async_copy(rel_hbm.at[i1], bbuf, sem1)
        cp3 = pltpu.async_copy(tt_hbm.at[i2], tbuf, sem2)
        cp1.wait()
        cp2.wait()
        cp3.wait()
        _rows_op(mbuf, EK, lambda i, j:
                 (abuf[i, pl.ds(j, L)] - bbuf[i, pl.ds(j, L)])
                 * tbuf[i, pl.ds(j, L)])
        pltpu.sync_copy(mbuf, aggS.at[i3], add=True)
        return 0
    lax.fori_loop(0, E_CHUNKS, _echunk, 0)

    q0 = w * Q_PER_W

    def _qchunk(ci, _):
        base = q0 + ci * EK
        pltpu.sync_copy(qedge_hbm.at[pl.ds(base, EK)], i0)
        cpA = pltpu.async_copy(dst_hbm.at[i0], i3, sem0)
        cpB = pltpu.async_copy(etime_hbm.at[i0], i2, sem1)
        pltpu.sync_copy(qr_hbm.at[pl.ds(base, EK)], i1)
        pltpu.sync_copy(qe_hbm.at[pl.ds(base, EK)], i4)
        cpA.wait()
        cpB.wait()
        cp1 = pltpu.async_copy(rel_hbm.at[i1], bbuf, sem0)
        cp2 = pltpu.async_copy(x0_hbm.at[i4], abuf, sem1)
        cp3 = pltpu.async_copy(tt_hbm.at[i2], tbuf, sem2)
        cp1.wait()
        cp2.wait()
        cp3.wait()
        _rows_op(mbuf, EK, lambda i, j:
                 -(abuf[i, pl.ds(j, L)] * bbuf[i, pl.ds(j, L)])
                 * tbuf[i, pl.ds(j, L)])
        pltpu.sync_copy(mbuf, aggS.at[i3], add=True)
        return 0
    lax.fori_loop(0, Q_CHUNKS, _qchunk, 0)

    plsc.subcore_barrier()

    def _oc(k, _):
        r = r0 + k * ZK
        pltpu.sync_copy(aggS.at[pl.ds(r, ZK)], mbuf)

        @pl.when(c == 0)
        def _():
            pltpu.sync_copy(mbuf, p0_hbm.at[pl.ds(r, ZK)])

        @pl.when(c == 1)
        def _():
            pltpu.sync_copy(mbuf, p1_hbm.at[pl.ds(r, ZK)])
        return 0
    lax.fori_loop(0, Z_CHUNKS, _oc, 0)


def _sc_agg():
  return pl.kernel(
    _sc_agg_body,
    out_type=(jax.ShapeDtypeStruct((NPAD, D), jnp.float32),
              jax.ShapeDtypeStruct((NPAD, D), jnp.float32)),
    mesh=_mesh(),
    scratch_types=[
        pltpu.VMEM((EK,), jnp.int32), pltpu.VMEM((EK,), jnp.int32),
        pltpu.VMEM((EK,), jnp.int32), pltpu.VMEM((EK,), jnp.int32),
        pltpu.VMEM((EK,), jnp.int32),
        pltpu.VMEM((EK, D), jnp.float32), pltpu.VMEM((EK, D), jnp.float32),
        pltpu.VMEM((EK, D), jnp.float32), pltpu.VMEM((EK, D), jnp.float32),
        pltpu.VMEM_SHARED((NPAD, D), jnp.float32),
        pltpu.SemaphoreType.DMA, pltpu.SemaphoreType.DMA,
        pltpu.SemaphoreType.DMA,
    ],
  )


def _sc_query_body(p0_hbm, p1_hbm, reltab_hbm, sub_hbm, rel_hbm,
                   subrows_hbm, relrows_hbm,
                   iS, iR, abuf, bbuf, mbuf, sem0, sem1):
    c = lax.axis_index("c")
    s = lax.axis_index("s")
    w = c * NS + s
    b0 = w * B_PER_W

    def _chunk(ci, _):
        base = b0 + ci * BK
        pltpu.sync_copy(sub_hbm.at[pl.ds(base, BK)], iS)
        pltpu.sync_copy(rel_hbm.at[pl.ds(base, BK)], iR)
        cp1 = pltpu.async_copy(p0_hbm.at[iS], abuf, sem0)
        cp2 = pltpu.async_copy(p1_hbm.at[iS], bbuf, sem1)
        cp1.wait()
        cp2.wait()
        _rows_op(mbuf, BK, lambda i, j:
                 abuf[i, pl.ds(j, L)] + bbuf[i, pl.ds(j, L)])
        pltpu.sync_copy(mbuf, subrows_hbm.at[pl.ds(base, BK)])
        cp3 = pltpu.async_copy(reltab_hbm.at[iR], abuf, sem0)
        cp3.wait()
        pltpu.sync_copy(abuf, relrows_hbm.at[pl.ds(base, BK)])
        return 0
    lax.fori_loop(0, B_CHUNKS, _chunk, 0)


def _sc_query():
  return pl.kernel(
    _sc_query_body,
    out_type=(jax.ShapeDtypeStruct((B, D), jnp.float32),
              jax.ShapeDtypeStruct((B, D), jnp.float32)),
    mesh=_mesh(),
    scratch_types=[
        pltpu.VMEM((BK,), jnp.int32), pltpu.VMEM((BK,), jnp.int32),
        pltpu.VMEM((BK, D), jnp.float32), pltpu.VMEM((BK, D), jnp.float32),
        pltpu.VMEM((BK, D), jnp.float32),
        pltpu.SemaphoreType.DMA, pltpu.SemaphoreType.DMA,
    ],
  )


def _tc_time_body(trange_ref, timef_ref, w_ref, phi_ref, tt_ref, temb_ref):
    tt_ref[...] = jnp.cos(trange_ref[...] * w_ref[...] + phi_ref[...])
    temb_ref[...] = jnp.cos(timef_ref[...] * w_ref[...] + phi_ref[...])


def _tc_x_body(p0_ref, p1_ref, w_ref, x_ref):
    acc = p0_ref[...] + p1_ref[...]
    x_ref[...] = jnp.tanh(
        jnp.dot(acc, w_ref[...], preferred_element_type=jnp.float32))


def _tc_q_body(subrows_ref, temb_ref, relrows_ref, w_ref, wpt_ref, wpb_ref,
               bproj_ref, wrel_ref, sub_emb_ref, rel_emb_ref):
    sx = jnp.tanh(jnp.dot(subrows_ref[...], w_ref[...],
                          preferred_element_type=jnp.float32))
    sub_emb_ref[...] = (
        jnp.dot(sx, wpt_ref[...], preferred_element_type=jnp.float32)
        + jnp.dot(temb_ref[...], wpb_ref[...],
                  preferred_element_type=jnp.float32)
        + bproj_ref[...])
    rel_emb_ref[...] = jnp.dot(relrows_ref[...], wrel_ref[...],
                               preferred_element_type=jnp.float32)


def kernel(x0, init_rel, W, w_rel, basis_freq, phase, W_proj, b_proj,
           edge_index, edge_type, edge_time, quals, sub, rel, time):
    src = edge_index[0]
    dst = edge_index[1]
    qr, qe, qedge = quals[0], quals[1], quals[2]

    trange = jnp.arange(TPAD, dtype=jnp.float32)[:, None]
    timef = time.astype(jnp.float32)[:, None]
    wrow = basis_freq[None, :]
    phirow = phase[None, :]

    tt, temb = pl.pallas_call(
        _tc_time_body,
        out_shape=(jax.ShapeDtypeStruct((TPAD, D), jnp.float32),
                   jax.ShapeDtypeStruct((B, D), jnp.float32)),
    )(trange, timef, wrow, phirow)

    p0, p1 = _sc_agg()(x0, init_rel, tt, src, dst, edge_type, edge_time,
                       qr, qe, qedge)

    subrows, relrows = _sc_query()(p0, p1, init_rel, sub, rel)

    bs = 1024
    x = pl.pallas_call(
        _tc_x_body,
        grid=(pl.cdiv(N, bs),),
        in_specs=[pl.BlockSpec((bs, D), lambda i: (i, 0)),
                  pl.BlockSpec((bs, D), lambda i: (i, 0)),
                  pl.BlockSpec((D, D), lambda i: (0, 0))],
        out_specs=pl.BlockSpec((bs, D), lambda i: (i, 0)),
        out_shape=jax.ShapeDtypeStruct((N, D), jnp.float32),
    )(p0, p1, W)

    sub_emb, rel_emb = pl.pallas_call(
        _tc_q_body,
        out_shape=(jax.ShapeDtypeStruct((B, D), jnp.float32),
                   jax.ShapeDtypeStruct((B, D), jnp.float32)),
    )(subrows, temb, relrows, W, W_proj[:D], W_proj[D:], b_proj[None, :],
      w_rel)

    return sub_emb, rel_emb, x, temb


# speedup vs baseline: 5.6465x; 1.7203x over previous
import functools

import jax
import jax.numpy as jnp
from jax import lax
from jax.experimental import pallas as pl
from jax.experimental.pallas import tpu as pltpu
from jax.experimental.pallas import tpu_sc as plsc

N = 10000
E = 320000
D = 128
NREL = 1000
NQ = 64000
B = 4096
TPAD = 368

NC = 2
NS = 16
NW = NC * NS
L = 16
VPR = D // L

EK = 40
E_PER_W = E // NW
E_CHUNKS = E_PER_W // EK
Q_PER_W = NQ // NW
Q_CHUNKS = Q_PER_W // EK
B_PER_W = B // NW
BK = 64
B_CHUNKS = B_PER_W // BK
NPAD = 10240
N_PER_S = NPAD // NS
ZK = 40
Z_CHUNKS = N_PER_S // ZK

def _mesh():
    return plsc.VectorSubcoreMesh(core_axis_name="c", subcore_axis_name="s",
                                  num_cores=NC, num_subcores=NS)


def _rows_op(dst_ref, k, body):
    def step(i, _):
        for jj in range(VPR):
            dst_ref[i, pl.ds(jj * L, L)] = body(i, jj * L)
        return 0
    lax.fori_loop(0, k, step, 0)


def _sc_agg_body(x0_hbm, rel_hbm, tt_hbm, src_hbm, dst_hbm, et_hbm, etime_hbm,
                 qr_hbm, qe_hbm, qedge_hbm,
                 p0_hbm, p1_hbm,
                 i0a, i1a, i2a, i3a, i0b, i1b, i2b, i3b, i4,
                 abufa, bbufa, tbufa, mbufa, abufb, bbufb, tbufb, mbufb,
                 aggS, semIa, semIb, semRa, semRb):
    c = lax.axis_index("c")
    s = lax.axis_index("s")
    w = c * NS + s

    seta = (i0a, i1a, i2a, i3a, abufa, bbufa, tbufa, mbufa, semIa, semRa)
    setb = (i0b, i1b, i2b, i3b, abufb, bbufb, tbufb, mbufb, semIb, semRb)

    zero16 = jnp.zeros((L,), jnp.float32)
    _rows_op(mbufa, ZK, lambda i, j: zero16)
    r0 = s * N_PER_S

    def _zc(k, _):
        pltpu.sync_copy(mbufa, aggS.at[pl.ds(r0 + k * ZK, ZK)])
        return 0
    lax.fori_loop(0, Z_CHUNKS, _zc, 0)
    plsc.subcore_barrier()

    e0 = w * E_PER_W

    def _issue_idx(ci, st):
        i0, i1, i2, i3 = st[0], st[1], st[2], st[3]
        base = e0 + ci * EK
        pltpu.async_copy(src_hbm.at[pl.ds(base, EK)], i0, st[8])
        pltpu.async_copy(et_hbm.at[pl.ds(base, EK)], i1, st[8])
        pltpu.async_copy(etime_hbm.at[pl.ds(base, EK)], i2, st[8])
        pltpu.async_copy(dst_hbm.at[pl.ds(base, EK)], i3, st[8])

    def _wait_idx(st):
        pltpu.make_async_copy(src_hbm.at[pl.ds(0, EK)], st[0], st[8]).wait()
        pltpu.make_async_copy(et_hbm.at[pl.ds(0, EK)], st[1], st[8]).wait()
        pltpu.make_async_copy(etime_hbm.at[pl.ds(0, EK)], st[2], st[8]).wait()
        pltpu.make_async_copy(dst_hbm.at[pl.ds(0, EK)], st[3], st[8]).wait()

    def _issue_rows(st):
        pltpu.async_copy(x0_hbm.at[st[0]], st[4], st[9])
        pltpu.async_copy(rel_hbm.at[st[1]], st[5], st[9])
        pltpu.async_copy(tt_hbm.at[st[2]], st[6], st[9])

    def _wait_rows(st):
        pltpu.make_async_copy(x0_hbm.at[st[0]], st[4], st[9]).wait()
        pltpu.make_async_copy(rel_hbm.at[st[1]], st[5], st[9]).wait()
        pltpu.make_async_copy(tt_hbm.at[st[2]], st[6], st[9]).wait()

    def _estep(cur, stS, stT):
        @pl.when(cur + 1 < E_CHUNKS)
        def _():
            _wait_idx(stT)
            _issue_rows(stT)
        _wait_rows(stS)
        a, b, t, m = stS[4], stS[5], stS[6], stS[7]
        _rows_op(m, EK, lambda i, j:
                 (a[i, pl.ds(j, L)] - b[i, pl.ds(j, L)]) * t[i, pl.ds(j, L)])
        pltpu.sync_copy(m, aggS.at[stS[3]], add=True)

        @pl.when(cur + 2 < E_CHUNKS)
        def _():
            _issue_idx(cur + 2, stS)

    _issue_idx(0, seta)
    _wait_idx(seta)
    _issue_rows(seta)
    _issue_idx(1, setb)

    def _epair(p, _):
        _estep(2 * p, seta, setb)
        _estep(2 * p + 1, setb, seta)
        return 0
    lax.fori_loop(0, E_CHUNKS // 2, _epair, 0)

    q0 = w * Q_PER_W

    def _qchunk(ci, _):
        base = q0 + ci * EK
        pltpu.sync_copy(qedge_hbm.at[pl.ds(base, EK)], i0a)
        cpA = pltpu.async_copy(dst_hbm.at[i0a], i3a, semIa)
        cpB = pltpu.async_copy(etime_hbm.at[i0a], i2a, semIb)
        pltpu.sync_copy(qr_hbm.at[pl.ds(base, EK)], i1a)
        pltpu.sync_copy(qe_hbm.at[pl.ds(base, EK)], i4)
        cpA.wait()
        cpB.wait()
        cp1 = pltpu.async_copy(rel_hbm.at[i1a], bbufa, semRa)
        cp2 = pltpu.async_copy(x0_hbm.at[i4], abufa, semRb)
        cp3 = pltpu.async_copy(tt_hbm.at[i2a], tbufa, semIa)
        cp1.wait()
        cp2.wait()
        cp3.wait()
        _rows_op(mbufa, EK, lambda i, j:
                 -(abufa[i, pl.ds(j, L)] * bbufa[i, pl.ds(j, L)])
                 * tbufa[i, pl.ds(j, L)])
        pltpu.sync_copy(mbufa, aggS.at[i3a], add=True)
        return 0
    lax.fori_loop(0, Q_CHUNKS, _qchunk, 0)

    plsc.subcore_barrier()

    def _oc(k, _):
        r = r0 + k * ZK
        pltpu.sync_copy(aggS.at[pl.ds(r, ZK)], mbufa)

        @pl.when(c == 0)
        def _():
            pltpu.sync_copy(mbufa, p0_hbm.at[pl.ds(r, ZK)])

        @pl.when(c == 1)
        def _():
            pltpu.sync_copy(mbufa, p1_hbm.at[pl.ds(r, ZK)])
        return 0
    lax.fori_loop(0, Z_CHUNKS, _oc, 0)


def _sc_agg():
  return pl.kernel(
    _sc_agg_body,
    out_type=(jax.ShapeDtypeStruct((NPAD, D), jnp.float32),
              jax.ShapeDtypeStruct((NPAD, D), jnp.float32)),
    mesh=_mesh(),
    scratch_types=[
        pltpu.VMEM((EK,), jnp.int32), pltpu.VMEM((EK,), jnp.int32),
        pltpu.VMEM((EK,), jnp.int32), pltpu.VMEM((EK,), jnp.int32),
        pltpu.VMEM((EK,), jnp.int32), pltpu.VMEM((EK,), jnp.int32),
        pltpu.VMEM((EK,), jnp.int32), pltpu.VMEM((EK,), jnp.int32),
        pltpu.VMEM((EK,), jnp.int32),
        pltpu.VMEM((EK, D), jnp.float32), pltpu.VMEM((EK, D), jnp.float32),
        pltpu.VMEM((EK, D), jnp.float32), pltpu.VMEM((EK, D), jnp.float32),
        pltpu.VMEM((EK, D), jnp.float32), pltpu.VMEM((EK, D), jnp.float32),
        pltpu.VMEM((EK, D), jnp.float32), pltpu.VMEM((EK, D), jnp.float32),
        pltpu.VMEM_SHARED((NPAD, D), jnp.float32),
        pltpu.SemaphoreType.DMA, pltpu.SemaphoreType.DMA,
        pltpu.SemaphoreType.DMA, pltpu.SemaphoreType.DMA,
    ],
  )


def _sc_query_body(p0_hbm, p1_hbm, reltab_hbm, sub_hbm, rel_hbm,
                   subrows_hbm, relrows_hbm,
                   iS, iR, abuf, bbuf, mbuf, sem0, sem1):
    c = lax.axis_index("c")
    s = lax.axis_index("s")
    w = c * NS + s
    b0 = w * B_PER_W

    def _chunk(ci, _):
        base = b0 + ci * BK
        pltpu.sync_copy(sub_hbm.at[pl.ds(base, BK)], iS)
        pltpu.sync_copy(rel_hbm.at[pl.ds(base, BK)], iR)
        cp1 = pltpu.async_copy(p0_hbm.at[iS], abuf, sem0)
        cp2 = pltpu.async_copy(p1_hbm.at[iS], bbuf, sem1)
        cp1.wait()
        cp2.wait()
        _rows_op(mbuf, BK, lambda i, j:
                 abuf[i, pl.ds(j, L)] + bbuf[i, pl.ds(j, L)])
        pltpu.sync_copy(mbuf, subrows_hbm.at[pl.ds(base, BK)])
        cp3 = pltpu.async_copy(reltab_hbm.at[iR], abuf, sem0)
        cp3.wait()
        pltpu.sync_copy(abuf, relrows_hbm.at[pl.ds(base, BK)])
        return 0
    lax.fori_loop(0, B_CHUNKS, _chunk, 0)


def _sc_query():
  return pl.kernel(
    _sc_query_body,
    out_type=(jax.ShapeDtypeStruct((B, D), jnp.float32),
              jax.ShapeDtypeStruct((B, D), jnp.float32)),
    mesh=_mesh(),
    scratch_types=[
        pltpu.VMEM((BK,), jnp.int32), pltpu.VMEM((BK,), jnp.int32),
        pltpu.VMEM((BK, D), jnp.float32), pltpu.VMEM((BK, D), jnp.float32),
        pltpu.VMEM((BK, D), jnp.float32),
        pltpu.SemaphoreType.DMA, pltpu.SemaphoreType.DMA,
    ],
  )


def _tc_time_body(trange_ref, timef_ref, w_ref, phi_ref, tt_ref, temb_ref):
    tt_ref[...] = jnp.cos(trange_ref[...] * w_ref[...] + phi_ref[...])
    temb_ref[...] = jnp.cos(timef_ref[...] * w_ref[...] + phi_ref[...])


def _tc_x_body(p0_ref, p1_ref, w_ref, x_ref):
    acc = p0_ref[...] + p1_ref[...]
    x_ref[...] = jnp.tanh(
        jnp.dot(acc, w_ref[...], preferred_element_type=jnp.float32))


def _tc_q_body(subrows_ref, temb_ref, relrows_ref, w_ref, wpt_ref, wpb_ref,
               bproj_ref, wrel_ref, sub_emb_ref, rel_emb_ref):
    sx = jnp.tanh(jnp.dot(subrows_ref[...], w_ref[...],
                          preferred_element_type=jnp.float32))
    sub_emb_ref[...] = (
        jnp.dot(sx, wpt_ref[...], preferred_element_type=jnp.float32)
        + jnp.dot(temb_ref[...], wpb_ref[...],
                  preferred_element_type=jnp.float32)
        + bproj_ref[...])
    rel_emb_ref[...] = jnp.dot(relrows_ref[...], wrel_ref[...],
                               preferred_element_type=jnp.float32)


def kernel(x0, init_rel, W, w_rel, basis_freq, phase, W_proj, b_proj,
           edge_index, edge_type, edge_time, quals, sub, rel, time):
    src = edge_index[0]
    dst = edge_index[1]
    qr, qe, qedge = quals[0], quals[1], quals[2]

    trange = jnp.arange(TPAD, dtype=jnp.float32)[:, None]
    timef = time.astype(jnp.float32)[:, None]
    wrow = basis_freq[None, :]
    phirow = phase[None, :]

    tt, temb = pl.pallas_call(
        _tc_time_body,
        out_shape=(jax.ShapeDtypeStruct((TPAD, D), jnp.float32),
                   jax.ShapeDtypeStruct((B, D), jnp.float32)),
    )(trange, timef, wrow, phirow)

    p0, p1 = _sc_agg()(x0, init_rel, tt, src, dst, edge_type, edge_time,
                       qr, qe, qedge)

    subrows, relrows = _sc_query()(p0, p1, init_rel, sub, rel)

    bs = 1024
    x = pl.pallas_call(
        _tc_x_body,
        grid=(pl.cdiv(N, bs),),
        in_specs=[pl.BlockSpec((bs, D), lambda i: (i, 0)),
                  pl.BlockSpec((bs, D), lambda i: (i, 0)),
                  pl.BlockSpec((D, D), lambda i: (0, 0))],
        out_specs=pl.BlockSpec((bs, D), lambda i: (i, 0)),
        out_shape=jax.ShapeDtypeStruct((N, D), jnp.float32),
    )(p0, p1, W)

    sub_emb, rel_emb = pl.pallas_call(
        _tc_q_body,
        out_shape=(jax.ShapeDtypeStruct((B, D), jnp.float32),
                   jax.ShapeDtypeStruct((B, D), jnp.float32)),
    )(subrows, temb, relrows, W, W_proj[:D], W_proj[D:], b_proj[None, :],
      w_rel)

    return sub_emb, rel_emb, x, temb


# speedup vs baseline: 6.3215x; 1.1196x over previous
import functools

import jax
import jax.numpy as jnp
from jax import lax
from jax.experimental import pallas as pl
from jax.experimental.pallas import tpu as pltpu
from jax.experimental.pallas import tpu_sc as plsc

N = 10000
E = 320000
D = 128
NREL = 1000
NQ = 64000
B = 4096
TPAD = 368

NC = 2
NS = 16
NW = NC * NS
L = 16
VPR = D // L

EK = 40
E_PER_W = E // NW
E_CHUNKS = E_PER_W // EK
Q_PER_W = NQ // NW
Q_CHUNKS = Q_PER_W // EK
B_PER_W = B // NW
BK = 64
B_CHUNKS = B_PER_W // BK
NPAD = 10240
N_PER_S = NPAD // NS
ZK = 40
Z_CHUNKS = N_PER_S // ZK

def _mesh():
    return plsc.VectorSubcoreMesh(core_axis_name="c", subcore_axis_name="s",
                                  num_cores=NC, num_subcores=NS)


def _rows_op(dst_ref, k, body):
    def step(i, _):
        for jj in range(VPR):
            dst_ref[i, pl.ds(jj * L, L)] = body(i, jj * L)
        return 0
    lax.fori_loop(0, k, step, 0)


def _sc_agg_body(x0_hbm, rel_hbm, tt_hbm, src_hbm, dst_hbm, et_hbm, etime_hbm,
                 qr_hbm, qe_hbm, qedge_hbm,
                 p0_hbm, p1_hbm,
                 i0a, i1a, i2a, i3a, i0b, i1b, i2b, i3b, i4, i5,
                 abufa, bbufa, tbufa, mbufa, abufb, bbufb, tbufb, mbufb,
                 aggS, semIa, semIb, semRa, semRb, semQa, semQb):
    c = lax.axis_index("c")
    s = lax.axis_index("s")
    w = c * NS + s

    seta = (i0a, i1a, i2a, i3a, abufa, bbufa, tbufa, mbufa, semIa, semRa)
    setb = (i0b, i1b, i2b, i3b, abufb, bbufb, tbufb, mbufb, semIb, semRb)

    zero16 = jnp.zeros((L,), jnp.float32)
    _rows_op(mbufa, ZK, lambda i, j: zero16)
    r0 = s * N_PER_S

    def _zc(k, _):
        pltpu.sync_copy(mbufa, aggS.at[pl.ds(r0 + k * ZK, ZK)])
        return 0
    lax.fori_loop(0, Z_CHUNKS, _zc, 0)
    plsc.subcore_barrier()

    e0 = w * E_PER_W

    def _issue_idx(ci, st):
        i0, i1, i2, i3 = st[0], st[1], st[2], st[3]
        base = e0 + ci * EK
        pltpu.async_copy(src_hbm.at[pl.ds(base, EK)], i0, st[8])
        pltpu.async_copy(et_hbm.at[pl.ds(base, EK)], i1, st[8])
        pltpu.async_copy(etime_hbm.at[pl.ds(base, EK)], i2, st[8])
        pltpu.async_copy(dst_hbm.at[pl.ds(base, EK)], i3, st[8])

    def _wait_idx(st):
        pltpu.make_async_copy(src_hbm.at[pl.ds(0, EK)], st[0], st[8]).wait()
        pltpu.make_async_copy(et_hbm.at[pl.ds(0, EK)], st[1], st[8]).wait()
        pltpu.make_async_copy(etime_hbm.at[pl.ds(0, EK)], st[2], st[8]).wait()
        pltpu.make_async_copy(dst_hbm.at[pl.ds(0, EK)], st[3], st[8]).wait()

    def _issue_rows(st):
        pltpu.async_copy(x0_hbm.at[st[0]], st[4], st[9])
        pltpu.async_copy(rel_hbm.at[st[1]], st[5], st[9])
        pltpu.async_copy(tt_hbm.at[st[2]], st[6], st[9])

    def _wait_rows(st):
        pltpu.make_async_copy(x0_hbm.at[st[0]], st[4], st[9]).wait()
        pltpu.make_async_copy(rel_hbm.at[st[1]], st[5], st[9]).wait()
        pltpu.make_async_copy(tt_hbm.at[st[2]], st[6], st[9]).wait()

    def _estep(cur, stS, stT):
        @pl.when(cur + 1 < E_CHUNKS)
        def _():
            _wait_idx(stT)
            _issue_rows(stT)
        _wait_rows(stS)
        a, b, t, m = stS[4], stS[5], stS[6], stS[7]
        _rows_op(m, EK, lambda i, j:
                 (a[i, pl.ds(j, L)] - b[i, pl.ds(j, L)]) * t[i, pl.ds(j, L)])
        pltpu.sync_copy(m, aggS.at[stS[3]], add=True)

        @pl.when(cur + 2 < E_CHUNKS)
        def _():
            _issue_idx(cur + 2, stS)

    _issue_idx(0, seta)
    _wait_idx(seta)
    _issue_rows(seta)
    _issue_idx(1, setb)

    def _epair(p, _):
        _estep(2 * p, seta, setb)
        _estep(2 * p + 1, setb, seta)
        return 0
    lax.fori_loop(0, E_CHUNKS // 2, _epair, 0)

    q0 = w * Q_PER_W
    qseta = (i0a, i1a, i2a, i3a, abufa, bbufa, tbufa, mbufa, semIa, semRa,
             i4, semQa)
    qsetb = (i0b, i1b, i2b, i3b, abufb, bbufb, tbufb, mbufb, semIb, semRb,
             i5, semQb)

    def _q_issue_qidx(ci, st):
        base = q0 + ci * EK
        pltpu.async_copy(qedge_hbm.at[pl.ds(base, EK)], st[0], st[11])
        pltpu.async_copy(qr_hbm.at[pl.ds(base, EK)], st[1], st[11])
        pltpu.async_copy(qe_hbm.at[pl.ds(base, EK)], st[10], st[11])

    def _q_wait_qidx(st):
        pltpu.make_async_copy(qedge_hbm.at[pl.ds(0, EK)], st[0], st[11]).wait()
        pltpu.make_async_copy(qr_hbm.at[pl.ds(0, EK)], st[1], st[11]).wait()
        pltpu.make_async_copy(qe_hbm.at[pl.ds(0, EK)], st[10], st[11]).wait()

    def _q_issue_scal(st):
        pltpu.async_copy(dst_hbm.at[st[0]], st[3], st[8])
        pltpu.async_copy(etime_hbm.at[st[0]], st[2], st[8])

    def _q_wait_scal(st):
        pltpu.make_async_copy(dst_hbm.at[st[0]], st[3], st[8]).wait()
        pltpu.make_async_copy(etime_hbm.at[st[0]], st[2], st[8]).wait()

    def _q_issue_rows(st):
        pltpu.async_copy(x0_hbm.at[st[10]], st[4], st[9])
        pltpu.async_copy(rel_hbm.at[st[1]], st[5], st[9])
        pltpu.async_copy(tt_hbm.at[st[2]], st[6], st[9])

    def _q_wait_rows(st):
        pltpu.make_async_copy(x0_hbm.at[st[10]], st[4], st[9]).wait()
        pltpu.make_async_copy(rel_hbm.at[st[1]], st[5], st[9]).wait()
        pltpu.make_async_copy(tt_hbm.at[st[2]], st[6], st[9]).wait()

    def _qstep(cur, stS, stT):
        @pl.when(cur + 1 < Q_CHUNKS)
        def _():
            _q_wait_qidx(stT)
            _q_issue_scal(stT)
        _q_wait_rows(stS)

        @pl.when(cur + 1 < Q_CHUNKS)
        def _():
            _q_wait_scal(stT)
            _q_issue_rows(stT)
        a, b, t, m = stS[4], stS[5], stS[6], stS[7]
        _rows_op(m, EK, lambda i, j:
                 -(a[i, pl.ds(j, L)] * b[i, pl.ds(j, L)]) * t[i, pl.ds(j, L)])
        pltpu.sync_copy(m, aggS.at[stS[3]], add=True)

        @pl.when(cur + 2 < Q_CHUNKS)
        def _():
            _q_issue_qidx(cur + 2, stS)

    _q_issue_qidx(0, qseta)
    _q_wait_qidx(qseta)
    _q_issue_scal(qseta)
    _q_wait_scal(qseta)
    _q_issue_rows(qseta)
    _q_issue_qidx(1, qsetb)

    def _qpair(p, _):
        _qstep(2 * p, qseta, qsetb)
        _qstep(2 * p + 1, qsetb, qseta)
        return 0
    lax.fori_loop(0, Q_CHUNKS // 2, _qpair, 0)

    plsc.subcore_barrier()

    def _oc(k, _):
        r = r0 + k * ZK
        pltpu.sync_copy(aggS.at[pl.ds(r, ZK)], mbufa)

        @pl.when(c == 0)
        def _():
            pltpu.sync_copy(mbufa, p0_hbm.at[pl.ds(r, ZK)])

        @pl.when(c == 1)
        def _():
            pltpu.sync_copy(mbufa, p1_hbm.at[pl.ds(r, ZK)])
        return 0
    lax.fori_loop(0, Z_CHUNKS, _oc, 0)


def _sc_agg():
  return pl.kernel(
    _sc_agg_body,
    out_type=(jax.ShapeDtypeStruct((NPAD, D), jnp.float32),
              jax.ShapeDtypeStruct((NPAD, D), jnp.float32)),
    mesh=_mesh(),
    scratch_types=[
        pltpu.VMEM((EK,), jnp.int32), pltpu.VMEM((EK,), jnp.int32),
        pltpu.VMEM((EK,), jnp.int32), pltpu.VMEM((EK,), jnp.int32),
        pltpu.VMEM((EK,), jnp.int32), pltpu.VMEM((EK,), jnp.int32),
        pltpu.VMEM((EK,), jnp.int32), pltpu.VMEM((EK,), jnp.int32),
        pltpu.VMEM((EK,), jnp.int32), pltpu.VMEM((EK,), jnp.int32),
        pltpu.VMEM((EK, D), jnp.float32), pltpu.VMEM((EK, D), jnp.float32),
        pltpu.VMEM((EK, D), jnp.float32), pltpu.VMEM((EK, D), jnp.float32),
        pltpu.VMEM((EK, D), jnp.float32), pltpu.VMEM((EK, D), jnp.float32),
        pltpu.VMEM((EK, D), jnp.float32), pltpu.VMEM((EK, D), jnp.float32),
        pltpu.VMEM_SHARED((NPAD, D), jnp.float32),
        pltpu.SemaphoreType.DMA, pltpu.SemaphoreType.DMA,
        pltpu.SemaphoreType.DMA, pltpu.SemaphoreType.DMA,
        pltpu.SemaphoreType.DMA, pltpu.SemaphoreType.DMA,
    ],
  )


def _sc_query_body(p0_hbm, p1_hbm, reltab_hbm, sub_hbm, rel_hbm,
                   subrows_hbm, relrows_hbm,
                   iS, iR, abuf, bbuf, mbuf, sem0, sem1):
    c = lax.axis_index("c")
    s = lax.axis_index("s")
    w = c * NS + s
    b0 = w * B_PER_W

    def _chunk(ci, _):
        base = b0 + ci * BK
        pltpu.sync_copy(sub_hbm.at[pl.ds(base, BK)], iS)
        pltpu.sync_copy(rel_hbm.at[pl.ds(base, BK)], iR)
        cp1 = pltpu.async_copy(p0_hbm.at[iS], abuf, sem0)
        cp2 = pltpu.async_copy(p1_hbm.at[iS], bbuf, sem1)
        cp1.wait()
        cp2.wait()
        _rows_op(mbuf, BK, lambda i, j:
                 abuf[i, pl.ds(j, L)] + bbuf[i, pl.ds(j, L)])
        pltpu.sync_copy(mbuf, subrows_hbm.at[pl.ds(base, BK)])
        cp3 = pltpu.async_copy(reltab_hbm.at[iR], abuf, sem0)
        cp3.wait()
        pltpu.sync_copy(abuf, relrows_hbm.at[pl.ds(base, BK)])
        return 0
    lax.fori_loop(0, B_CHUNKS, _chunk, 0)


def _sc_query():
  return pl.kernel(
    _sc_query_body,
    out_type=(jax.ShapeDtypeStruct((B, D), jnp.float32),
              jax.ShapeDtypeStruct((B, D), jnp.float32)),
    mesh=_mesh(),
    scratch_types=[
        pltpu.VMEM((BK,), jnp.int32), pltpu.VMEM((BK,), jnp.int32),
        pltpu.VMEM((BK, D), jnp.float32), pltpu.VMEM((BK, D), jnp.float32),
        pltpu.VMEM((BK, D), jnp.float32),
        pltpu.SemaphoreType.DMA, pltpu.SemaphoreType.DMA,
    ],
  )


def _tc_time_body(trange_ref, timef_ref, w_ref, phi_ref, tt_ref, temb_ref):
    tt_ref[...] = jnp.cos(trange_ref[...] * w_ref[...] + phi_ref[...])
    temb_ref[...] = jnp.cos(timef_ref[...] * w_ref[...] + phi_ref[...])


def _tc_x_body(p0_ref, p1_ref, w_ref, x_ref):
    acc = p0_ref[...] + p1_ref[...]
    x_ref[...] = jnp.tanh(
        jnp.dot(acc, w_ref[...], preferred_element_type=jnp.float32))


def _tc_q_body(subrows_ref, temb_ref, relrows_ref, w_ref, wpt_ref, wpb_ref,
               bproj_ref, wrel_ref, sub_emb_ref, rel_emb_ref):
    sx = jnp.tanh(jnp.dot(subrows_ref[...], w_ref[...],
                          preferred_element_type=jnp.float32))
    sub_emb_ref[...] = (
        jnp.dot(sx, wpt_ref[...], preferred_element_type=jnp.float32)
        + jnp.dot(temb_ref[...], wpb_ref[...],
                  preferred_element_type=jnp.float32)
        + bproj_ref[...])
    rel_emb_ref[...] = jnp.dot(relrows_ref[...], wrel_ref[...],
                               preferred_element_type=jnp.float32)


def kernel(x0, init_rel, W, w_rel, basis_freq, phase, W_proj, b_proj,
           edge_index, edge_type, edge_time, quals, sub, rel, time):
    src = edge_index[0]
    dst = edge_index[1]
    qr, qe, qedge = quals[0], quals[1], quals[2]

    trange = jnp.arange(TPAD, dtype=jnp.float32)[:, None]
    timef = time.astype(jnp.float32)[:, None]
    wrow = basis_freq[None, :]
    phirow = phase[None, :]

    tt, temb = pl.pallas_call(
        _tc_time_body,
        out_shape=(jax.ShapeDtypeStruct((TPAD, D), jnp.float32),
                   jax.ShapeDtypeStruct((B, D), jnp.float32)),
    )(trange, timef, wrow, phirow)

    p0, p1 = _sc_agg()(x0, init_rel, tt, src, dst, edge_type, edge_time,
                       qr, qe, qedge)

    subrows, relrows = _sc_query()(p0, p1, init_rel, sub, rel)

    bs = 1024
    x = pl.pallas_call(
        _tc_x_body,
        grid=(pl.cdiv(N, bs),),
        in_specs=[pl.BlockSpec((bs, D), lambda i: (i, 0)),
                  pl.BlockSpec((bs, D), lambda i: (i, 0)),
                  pl.BlockSpec((D, D), lambda i: (0, 0))],
        out_specs=pl.BlockSpec((bs, D), lambda i: (i, 0)),
        out_shape=jax.ShapeDtypeStruct((N, D), jnp.float32),
    )(p0, p1, W)

    sub_emb, rel_emb = pl.pallas_call(
        _tc_q_body,
        out_shape=(jax.ShapeDtypeStruct((B, D), jnp.float32),
                   jax.ShapeDtypeStruct((B, D), jnp.float32)),
    )(subrows, temb, relrows, W, W_proj[:D], W_proj[D:], b_proj[None, :],
      w_rel)

    return sub_emb, rel_emb, x, temb
